# baseline (speedup 1.0000x reference)
import functools

import jax
import jax.numpy as jnp
from jax import lax
from jax.experimental import pallas as pl
from jax.experimental.pallas import tpu as pltpu
from jax.experimental.pallas import tpu_sc as plsc

N = 10000
E = 320000
D = 128

NC = 2
NS = 16
NW = NC * NS
C = 128
NCHUNK = 80
EPW = NCHUNK * C
EP = NW * EPW
NP = 10240
RPT = NP // NS
SHIFT = 14
MASK = (1 << SHIFT) - 1


def _sc_agg(x, packed3):
    mesh = plsc.VectorSubcoreMesh(core_axis_name="c", subcore_axis_name="s")

    @functools.partial(
        pl.kernel, mesh=mesh,
        out_type=jax.ShapeDtypeStruct((NC, NP, D), jnp.float32),
        scratch_types=[
            pltpu.VMEM((NCHUNK, C), jnp.int32),
            pltpu.VMEM((C,), jnp.int32),
            pltpu.VMEM((C,), jnp.int32),
            pltpu.VMEM((C,), jnp.int32),
            pltpu.VMEM((C,), jnp.int32),
            pltpu.VMEM((C, D), jnp.float32),
            pltpu.VMEM((C, D), jnp.float32),
            pltpu.VMEM_SHARED((NP, D), jnp.float32),
            pltpu.SemaphoreType.DMA,
            pltpu.SemaphoreType.DMA,
            pltpu.SemaphoreType.DMA,
            pltpu.SemaphoreType.DMA,
        ],
        compiler_params=pltpu.CompilerParams(needs_layout_passes=False))
    def k(x_hbm, packed_hbm, agg_out, packed_v, src_a, dst_a, src_b, dst_b,
          rows_a, rows_b, acc, sem_a, sem_b, ssem_a, ssem_b):
        cid = lax.axis_index("c")
        sid = lax.axis_index("s")
        wid = sid * NC + cid

        pltpu.sync_copy(packed_hbm.at[wid], packed_v)

        def zb(i, carry):
            for j in range(D // 16):
                rows_a[i, pl.ds(j * 16, 16)] = jnp.zeros((16,), jnp.float32)
            return carry
        lax.fori_loop(0, C, zb, 0)
        for t in range(RPT // C):
            pltpu.sync_copy(rows_a, acc.at[pl.ds(sid * RPT + t * C, C)])
        plsc.subcore_barrier()

        def unpack(i, src_c, dst_c):
            for k2 in range(C // 16):
                w = packed_v[i, pl.ds(k2 * 16, 16)]
                src_c[pl.ds(k2 * 16, 16)] = lax.shift_right_logical(w, SHIFT)
                dst_c[pl.ds(k2 * 16, 16)] = lax.bitwise_and(w, MASK)

        def gather(buf, src_c, sem):
            pltpu.async_copy(x_hbm.at[src_c], buf, sem)

        def gwait(buf, src_c, sem):
            pltpu.make_async_copy(x_hbm.at[src_c], buf, sem).wait()

        def scat(buf, dst_c, sem):
            pltpu.async_copy(buf, acc.at[dst_c], sem, add=True)

        def swait(buf, dst_c, sem):
            pltpu.make_async_copy(buf, acc.at[dst_c], sem).wait()

        unpack(0, src_a, dst_a)
        gather(rows_a, src_a, sem_a)
        unpack(1, src_b, dst_b)
        gather(rows_b, src_b, sem_b)

        def body(j, carry):
            i0 = 2 * j
            gwait(rows_a, src_a, sem_a)
            scat(rows_a, dst_a, ssem_a)
            gwait(rows_b, src_b, sem_b)
            scat(rows_b, dst_b, ssem_b)
            swait(rows_a, dst_a, ssem_a)
            @pl.when(i0 + 2 < NCHUNK)
            def _():
                unpack(i0 + 2, src_a, dst_a)
                gather(rows_a, src_a, sem_a)
            swait(rows_b, dst_b, ssem_b)
            @pl.when(i0 + 3 < NCHUNK)
            def _():
                unpack(i0 + 3, src_b, dst_b)
                gather(rows_b, src_b, sem_b)
            return carry
        lax.fori_loop(0, NCHUNK // 2, body, 0)
        plsc.subcore_barrier()

        pltpu.sync_copy(acc.at[pl.ds(sid * RPT, RPT)],
                        agg_out.at[cid, pl.ds(sid * RPT, RPT)])

    return k(x, packed3)


def _sc_cnt(packed3):
    mesh = plsc.VectorSubcoreMesh(core_axis_name="c", subcore_axis_name="s")

    @functools.partial(
        pl.kernel, mesh=mesh,
        out_type=jax.ShapeDtypeStruct((NW * NP,), jnp.float32),
        scratch_types=[
            pltpu.VMEM((NCHUNK, C), jnp.int32),
            pltpu.VMEM((NP,), jnp.float32),
        ],
        compiler_params=pltpu.CompilerParams(needs_layout_passes=False))
    def k(packed_hbm, cnt_out, packed_v, cnt_t):
        cid = lax.axis_index("c")
        sid = lax.axis_index("s")
        wid = sid * NC + cid
        pltpu.sync_copy(packed_hbm.at[wid], packed_v)
        def zc(i, carry):
            cnt_t[pl.ds(i * 16, 16)] = jnp.zeros((16,), jnp.float32)
            return carry
        lax.fori_loop(0, NP // 16, zc, 0)
        ones16 = jnp.ones((16,), jnp.float32)
        def body(i, carry):
            for k2 in range(C // 16):
                idx = lax.bitwise_and(packed_v[i, pl.ds(k2 * 16, 16)], MASK)
                plsc.addupdate_scatter(cnt_t, [idx], ones16)
            return carry
        lax.fori_loop(0, NCHUNK, body, 0)
        pltpu.sync_copy(cnt_t, cnt_out.at[pl.ds(wid * NP, NP)])

    return k(packed3)


def _combine_body(p_ref, c_ref, x_ref, wl_ref, b_ref, wr_ref, o_ref, *, relu):
    cnt = jnp.maximum(jnp.sum(c_ref[:], axis=0), 1.0)[:, None]
    mean = (p_ref[0] + p_ref[1]) / cnt
    dn = (((1,), (1,)), ((), ()))
    y = lax.dot_general(mean, wl_ref[:], dn,
                        preferred_element_type=jnp.float32)
    y = y + b_ref[:]
    y = y + lax.dot_general(x_ref[:], wr_ref[:], dn,
                            preferred_element_type=jnp.float32)
    o_ref[:] = jnp.maximum(y, 0.0) if relu else y


def _tc_combine(p, c, x, W_l, b_l, W_r, relu):
    R = 1024
    grid = (NP // R,)
    return pl.pallas_call(
        functools.partial(_combine_body, relu=relu),
        grid=grid,
        in_specs=[
            pl.BlockSpec((NC, R, D), lambda i: (0, i, 0)),
            pl.BlockSpec((NW, R), lambda i: (0, i)),
            pl.BlockSpec((R, D), lambda i: (i, 0)),
            pl.BlockSpec((D, D), lambda i: (0, 0)),
            pl.BlockSpec((1, D), lambda i: (0, 0)),
            pl.BlockSpec((D, D), lambda i: (0, 0)),
        ],
        out_specs=pl.BlockSpec((R, D), lambda i: (i, 0)),
        out_shape=jax.ShapeDtypeStruct((NP, D), jnp.float32),
    )(p, c.reshape(NW, NP), x, W_l, b_l.reshape(1, D), W_r)


def _pack_edges(edge_index):
    src = edge_index[0]
    dst = edge_index[1]
    pad = EP - E
    src = jnp.concatenate([src, jnp.zeros((pad,), jnp.int32)])
    dst = jnp.concatenate([dst, jnp.full((pad,), N, jnp.int32)])
    packed = jnp.left_shift(src, SHIFT) | dst
    return packed.reshape(NW, NCHUNK, C)


def kernel(x, edge_index, W1_l, b1_l, W1_r, W2_l, b2_l, W2_r):
    packed = _pack_edges(edge_index)
    x_p = jnp.pad(x, ((0, NP - N), (0, 0)))
    cnt = _sc_cnt(packed)
    agg1 = _sc_agg(x, packed)
    h = _tc_combine(agg1, cnt, x_p, W1_l, b1_l, W1_r, relu=True)
    agg2 = _sc_agg(h, packed)
    out = _tc_combine(agg2, cnt, h, W2_l, b2_l, W2_r, relu=False)
    return out[:N]


# speedup vs baseline: 1.4182x; 1.4182x over previous
import functools

import jax
import jax.numpy as jnp
from jax import lax
from jax.experimental import pallas as pl
from jax.experimental.pallas import tpu as pltpu
from jax.experimental.pallas import tpu_sc as plsc

N = 10000
E = 320000
D = 128

NC = 2
NS = 16
NW = NC * NS
EPW = E // NW
C = 80
NCHUNK = EPW // C
NP = 10240
RPT = NP // NS


def _sc_agg(x, src, dst, with_cnt):
    mesh = plsc.VectorSubcoreMesh(core_axis_name="c", subcore_axis_name="s")

    out_type = [jax.ShapeDtypeStruct((NC, NP, D), jnp.float32)]
    scratch = [
        pltpu.VMEM((C,), jnp.int32),
        pltpu.VMEM((C,), jnp.int32),
        pltpu.VMEM((C, D), jnp.float32),
        pltpu.VMEM_SHARED((NP, D), jnp.float32),
        pltpu.SemaphoreType.DMA,
    ]
    if with_cnt:
        out_type.append(jax.ShapeDtypeStruct((NW * NP,), jnp.float32))
        scratch.append(pltpu.VMEM((NP,), jnp.float32))

    @functools.partial(
        pl.kernel, mesh=mesh, out_type=out_type, scratch_types=scratch,
        compiler_params=pltpu.CompilerParams(needs_layout_passes=False))
    def k(x_hbm, src_hbm, dst_hbm, *refs):
        if with_cnt:
            agg_out, cnt_out, src_v, dst_v, rows_v, acc, sem, cnt_t = refs
        else:
            agg_out, src_v, dst_v, rows_v, acc, sem = refs
        cid = lax.axis_index("c")
        sid = lax.axis_index("s")
        wid = sid * NC + cid

        def zb(i, carry):
            for j in range(D // 16):
                rows_v[i, pl.ds(j * 16, 16)] = jnp.zeros((16,), jnp.float32)
            return carry
        lax.fori_loop(0, C, zb, 0)
        for t in range(RPT // C):
            pltpu.sync_copy(rows_v, acc.at[pl.ds(sid * RPT + t * C, C)])
        if with_cnt:
            def zc(i, carry):
                cnt_t[pl.ds(i * 16, 16)] = jnp.zeros((16,), jnp.float32)
                return carry
            lax.fori_loop(0, NP // 16, zc, 0)
        plsc.subcore_barrier()

        def body(i, carry):
            base = wid * EPW + i * C
            pltpu.sync_copy(src_hbm.at[pl.ds(base, C)], src_v)
            pltpu.sync_copy(dst_hbm.at[pl.ds(base, C)], dst_v)
            pltpu.async_copy(x_hbm.at[src_v], rows_v, sem).wait()
            pltpu.sync_copy(rows_v, acc.at[dst_v], add=True)
            if with_cnt:
                ones16 = jnp.ones((16,), jnp.float32)
                for k2 in range(C // 16):
                    idx = dst_v[pl.ds(k2 * 16, 16)]
                    plsc.addupdate_scatter(cnt_t, [idx], ones16)
            return carry
        lax.fori_loop(0, NCHUNK, body, 0)
        plsc.subcore_barrier()

        pltpu.sync_copy(acc.at[pl.ds(sid * RPT, RPT)],
                        agg_out.at[cid, pl.ds(sid * RPT, RPT)])
        if with_cnt:
            pltpu.sync_copy(cnt_t, cnt_out.at[pl.ds(wid * NP, NP)])

    return k(x, src, dst)


def _combine_body(p_ref, c_ref, x_ref, wl_ref, b_ref, wr_ref, o_ref, *, relu):
    cnt = jnp.maximum(jnp.sum(c_ref[:], axis=0), 1.0)[:, None]
    mean = (p_ref[0] + p_ref[1]) / cnt
    dn = (((1,), (1,)), ((), ()))
    y = lax.dot_general(mean, wl_ref[:], dn,
                        preferred_element_type=jnp.float32)
    y = y + b_ref[:]
    y = y + lax.dot_general(x_ref[:], wr_ref[:], dn,
                            preferred_element_type=jnp.float32)
    o_ref[:] = jnp.maximum(y, 0.0) if relu else y


def _tc_combine(p, c, x, W_l, b_l, W_r, relu):
    R = 1024
    grid = (NP // R,)
    return pl.pallas_call(
        functools.partial(_combine_body, relu=relu),
        grid=grid,
        in_specs=[
            pl.BlockSpec((NC, R, D), lambda i: (0, i, 0)),
            pl.BlockSpec((NW, R), lambda i: (0, i)),
            pl.BlockSpec((R, D), lambda i: (i, 0)),
            pl.BlockSpec((D, D), lambda i: (0, 0)),
            pl.BlockSpec((1, D), lambda i: (0, 0)),
            pl.BlockSpec((D, D), lambda i: (0, 0)),
        ],
        out_specs=pl.BlockSpec((R, D), lambda i: (i, 0)),
        out_shape=jax.ShapeDtypeStruct((NP, D), jnp.float32),
    )(p, c.reshape(NW, NP), x, W_l, b_l.reshape(1, D), W_r)


def kernel(x, edge_index, W1_l, b1_l, W1_r, W2_l, b2_l, W2_r):
    src = edge_index[0]
    dst = edge_index[1]
    x_p = jnp.pad(x, ((0, NP - N), (0, 0)))
    agg1, cnt = _sc_agg(x, src, dst, with_cnt=True)
    h = _tc_combine(agg1, cnt, x_p, W1_l, b1_l, W1_r, relu=True)
    (agg2,) = _sc_agg(h, src, dst, with_cnt=False)
    out = _tc_combine(agg2, cnt, h, W2_l, b2_l, W2_r, relu=False)
    return out[:N]


# speedup vs baseline: 2.2686x; 1.5997x over previous
import functools

import jax
import jax.numpy as jnp
from jax import lax
from jax.experimental import pallas as pl
from jax.experimental.pallas import tpu as pltpu
from jax.experimental.pallas import tpu_sc as plsc

N = 10000
E = 320000
D = 128

NC = 2
NS = 16
NW = NC * NS
EPW = E // NW
C = 80
NCHUNK = EPW // C
NP = 10240
RPT = NP // NS


def _sc_agg(x, src, dst, with_cnt):
    mesh = plsc.VectorSubcoreMesh(core_axis_name="c", subcore_axis_name="s")

    out_type = [jax.ShapeDtypeStruct((NC, NP, D), jnp.float32)]
    scratch = [
        pltpu.VMEM((C,), jnp.int32),
        pltpu.VMEM((C,), jnp.int32),
        pltpu.VMEM((C,), jnp.int32),
        pltpu.VMEM((C,), jnp.int32),
        pltpu.VMEM((C, D), jnp.float32),
        pltpu.VMEM((C, D), jnp.float32),
        pltpu.VMEM_SHARED((NP, D), jnp.float32),
        pltpu.SemaphoreType.DMA,
        pltpu.SemaphoreType.DMA,
        pltpu.SemaphoreType.DMA,
        pltpu.SemaphoreType.DMA,
    ]
    if with_cnt:
        out_type.append(jax.ShapeDtypeStruct((NW * NP,), jnp.float32))
        scratch.append(pltpu.VMEM((NP,), jnp.float32))

    @functools.partial(
        pl.kernel, mesh=mesh, out_type=out_type, scratch_types=scratch,
        compiler_params=pltpu.CompilerParams(needs_layout_passes=False))
    def k(x_hbm, src_hbm, dst_hbm, *refs):
        if with_cnt:
            (agg_out, cnt_out, src_a, dst_a, src_b, dst_b, rows_a, rows_b,
             acc, gsem_a, gsem_b, ssem_a, ssem_b, cnt_t) = refs
        else:
            (agg_out, src_a, dst_a, src_b, dst_b, rows_a, rows_b,
             acc, gsem_a, gsem_b, ssem_a, ssem_b) = refs
        cid = lax.axis_index("c")
        sid = lax.axis_index("s")
        wid = sid * NC + cid

        def zb(i, carry):
            for j in range(D // 16):
                rows_a[i, pl.ds(j * 16, 16)] = jnp.zeros((16,), jnp.float32)
            return carry
        lax.fori_loop(0, C, zb, 0)
        for t in range(RPT // C):
            pltpu.sync_copy(rows_a, acc.at[pl.ds(sid * RPT + t * C, C)])
        if with_cnt:
            def zc(i, carry):
                cnt_t[pl.ds(i * 16, 16)] = jnp.zeros((16,), jnp.float32)
                return carry
            lax.fori_loop(0, NP // 16, zc, 0)
        plsc.subcore_barrier()

        base0 = wid * EPW

        def stage(i, src_c, dst_c):
            pltpu.sync_copy(src_hbm.at[pl.ds(base0 + i * C, C)], src_c)
            pltpu.sync_copy(dst_hbm.at[pl.ds(base0 + i * C, C)], dst_c)

        def gather(buf, src_c, sem):
            pltpu.async_copy(x_hbm.at[src_c], buf, sem)

        def gwait(buf, src_c, sem):
            pltpu.make_async_copy(x_hbm.at[src_c], buf, sem).wait()

        def scat(buf, dst_c, sem):
            pltpu.async_copy(buf, acc.at[dst_c], sem, add=True)

        def swait(buf, dst_c, sem):
            pltpu.make_async_copy(buf, acc.at[dst_c], sem).wait()

        def count(dst_c):
            if with_cnt:
                ones16 = jnp.ones((16,), jnp.float32)
                for k2 in range(C // 16):
                    idx = dst_c[pl.ds(k2 * 16, 16)]
                    plsc.addupdate_scatter(cnt_t, [idx], ones16)

        stage(0, src_a, dst_a)
        gather(rows_a, src_a, gsem_a)

        def body(j, carry):
            i0 = 2 * j
            stage(i0 + 1, src_b, dst_b)
            gather(rows_b, src_b, gsem_b)
            gwait(rows_a, src_a, gsem_a)
            scat(rows_a, dst_a, ssem_a)
            count(dst_a)
            swait(rows_a, dst_a, ssem_a)
            stage(i0 + 2, src_a, dst_a)
            gather(rows_a, src_a, gsem_a)
            gwait(rows_b, src_b, gsem_b)
            scat(rows_b, dst_b, ssem_b)
            count(dst_b)
            swait(rows_b, dst_b, ssem_b)
            return carry
        lax.fori_loop(0, NCHUNK // 2, body, 0)
        gwait(rows_a, src_a, gsem_a)
        pltpu.sync_copy(rows_a, acc.at[dst_a], add=True)
        count(dst_a)
        plsc.subcore_barrier()

        pltpu.sync_copy(acc.at[pl.ds(sid * RPT, RPT)],
                        agg_out.at[cid, pl.ds(sid * RPT, RPT)])
        if with_cnt:
            pltpu.sync_copy(cnt_t, cnt_out.at[pl.ds(wid * NP, NP)])

    return k(x, src, dst)


def _combine_body(p_ref, c_ref, x_ref, wl_ref, b_ref, wr_ref, o_ref, *, relu):
    cnt = jnp.maximum(jnp.sum(c_ref[:], axis=0), 1.0)[:, None]
    mean = (p_ref[0] + p_ref[1]) / cnt
    dn = (((1,), (1,)), ((), ()))
    y = lax.dot_general(mean, wl_ref[:], dn,
                        preferred_element_type=jnp.float32)
    y = y + b_ref[:]
    y = y + lax.dot_general(x_ref[:], wr_ref[:], dn,
                            preferred_element_type=jnp.float32)
    o_ref[:] = jnp.maximum(y, 0.0) if relu else y


def _tc_combine(p, c, x, W_l, b_l, W_r, relu):
    R = 1024
    grid = (NP // R,)
    return pl.pallas_call(
        functools.partial(_combine_body, relu=relu),
        grid=grid,
        in_specs=[
            pl.BlockSpec((NC, R, D), lambda i: (0, i, 0)),
            pl.BlockSpec((NW, R), lambda i: (0, i)),
            pl.BlockSpec((R, D), lambda i: (i, 0)),
            pl.BlockSpec((D, D), lambda i: (0, 0)),
            pl.BlockSpec((1, D), lambda i: (0, 0)),
            pl.BlockSpec((D, D), lambda i: (0, 0)),
        ],
        out_specs=pl.BlockSpec((R, D), lambda i: (i, 0)),
        out_shape=jax.ShapeDtypeStruct((NP, D), jnp.float32),
    )(p, c.reshape(NW, NP), x, W_l, b_l.reshape(1, D), W_r)


def kernel(x, edge_index, W1_l, b1_l, W1_r, W2_l, b2_l, W2_r):
    src = edge_index[0]
    dst = edge_index[1]
    x_p = jnp.pad(x, ((0, NP - N), (0, 0)))
    agg1, cnt = _sc_agg(x, src, dst, with_cnt=True)
    h = _tc_combine(agg1, cnt, x_p, W1_l, b1_l, W1_r, relu=True)
    (agg2,) = _sc_agg(h, src, dst, with_cnt=False)
    out = _tc_combine(agg2, cnt, h, W2_l, b2_l, W2_r, relu=False)
    return out[:N]


# speedup vs baseline: 2.5808x; 1.1376x over previous
import functools

import jax
import jax.numpy as jnp
from jax import lax
from jax.experimental import pallas as pl
from jax.experimental.pallas import tpu as pltpu
from jax.experimental.pallas import tpu_sc as plsc

N = 10000
E = 320000
D = 128

NC = 2
NS = 16
NW = NC * NS
EPW = E // NW
C = 80
NCHUNK = EPW // C
NP = 10240
RPT = NP // NS


def _sc_agg(x, src, dst, with_cnt):
    mesh = plsc.VectorSubcoreMesh(core_axis_name="c", subcore_axis_name="s")

    out_type = [jax.ShapeDtypeStruct((NC, NP, D), jnp.float32)]
    scratch = [
        pltpu.VMEM((C,), jnp.int32),
        pltpu.VMEM((C,), jnp.int32),
        pltpu.VMEM((C,), jnp.int32),
        pltpu.VMEM((C,), jnp.int32),
        pltpu.VMEM((C, D), jnp.float32),
        pltpu.VMEM((C, D), jnp.float32),
        pltpu.VMEM_SHARED((NP, D), jnp.float32),
        pltpu.SemaphoreType.DMA,
        pltpu.SemaphoreType.DMA,
        pltpu.SemaphoreType.DMA,
        pltpu.SemaphoreType.DMA,
        pltpu.SemaphoreType.DMA,
        pltpu.SemaphoreType.DMA,
    ]
    if with_cnt:
        out_type.append(jax.ShapeDtypeStruct((NW * NP,), jnp.float32))
        scratch.append(pltpu.VMEM((NP,), jnp.float32))

    @functools.partial(
        pl.kernel, mesh=mesh, out_type=out_type, scratch_types=scratch,
        compiler_params=pltpu.CompilerParams(needs_layout_passes=False))
    def k(x_hbm, src_hbm, dst_hbm, *refs):
        if with_cnt:
            (agg_out, cnt_out, src_a, dst_a, src_b, dst_b, rows_a, rows_b,
             acc, gsem_a, gsem_b, ssem_a, ssem_b, isem_a, isem_b,
             cnt_t) = refs
        else:
            (agg_out, src_a, dst_a, src_b, dst_b, rows_a, rows_b,
             acc, gsem_a, gsem_b, ssem_a, ssem_b, isem_a, isem_b) = refs
        cid = lax.axis_index("c")
        sid = lax.axis_index("s")
        wid = sid * NC + cid

        def zb(i, carry):
            for j in range(D // 16):
                rows_a[i, pl.ds(j * 16, 16)] = jnp.zeros((16,), jnp.float32)
            return carry
        lax.fori_loop(0, C, zb, 0)
        for t in range(RPT // C):
            pltpu.sync_copy(rows_a, acc.at[pl.ds(sid * RPT + t * C, C)])
        if with_cnt:
            def zc(i, carry):
                cnt_t[pl.ds(i * 16, 16)] = jnp.zeros((16,), jnp.float32)
                return carry
            lax.fori_loop(0, NP // 16, zc, 0)
        plsc.subcore_barrier()

        base0 = wid * EPW

        def stage(i, src_c, dst_c, sem):
            pltpu.async_copy(src_hbm.at[pl.ds(base0 + i * C, C)], src_c, sem)
            pltpu.async_copy(dst_hbm.at[pl.ds(base0 + i * C, C)], dst_c, sem)

        def iwait(i, src_c, dst_c, sem):
            pltpu.make_async_copy(
                src_hbm.at[pl.ds(base0 + i * C, C)], src_c, sem).wait()
            pltpu.make_async_copy(
                dst_hbm.at[pl.ds(base0 + i * C, C)], dst_c, sem).wait()

        def gather(buf, src_c, sem):
            pltpu.async_copy(x_hbm.at[src_c], buf, sem)

        def gwait(buf, src_c, sem):
            pltpu.make_async_copy(x_hbm.at[src_c], buf, sem).wait()

        def scat(buf, dst_c, sem):
            pltpu.async_copy(buf, acc.at[dst_c], sem, add=True)

        def swait(buf, dst_c, sem):
            pltpu.make_async_copy(buf, acc.at[dst_c], sem).wait()

        def count(dst_c):
            if with_cnt:
                ones16 = jnp.ones((16,), jnp.float32)
                for k2 in range(C // 16):
                    idx = dst_c[pl.ds(k2 * 16, 16)]
                    plsc.addupdate_scatter(cnt_t, [idx], ones16)

        stage(0, src_a, dst_a, isem_a)
        iwait(0, src_a, dst_a, isem_a)
        gather(rows_a, src_a, gsem_a)
        stage(1, src_b, dst_b, isem_b)

        def body(j, carry):
            i0 = 2 * j
            gwait(rows_a, src_a, gsem_a)
            scat(rows_a, dst_a, ssem_a)
            count(dst_a)
            iwait(i0 + 1, src_b, dst_b, isem_b)
            gather(rows_b, src_b, gsem_b)
            swait(rows_a, dst_a, ssem_a)
            stage(i0 + 2, src_a, dst_a, isem_a)
            gwait(rows_b, src_b, gsem_b)
            scat(rows_b, dst_b, ssem_b)
            count(dst_b)
            iwait(i0 + 2, src_a, dst_a, isem_a)
            gather(rows_a, src_a, gsem_a)
            swait(rows_b, dst_b, ssem_b)
            @pl.when(j + 1 < NCHUNK // 2)
            def _():
                stage(i0 + 3, src_b, dst_b, isem_b)
            return carry
        lax.fori_loop(0, NCHUNK // 2, body, 0)
        gwait(rows_a, src_a, gsem_a)
        pltpu.sync_copy(rows_a, acc.at[dst_a], add=True)
        count(dst_a)
        plsc.subcore_barrier()

        pltpu.sync_copy(acc.at[pl.ds(sid * RPT, RPT)],
                        agg_out.at[cid, pl.ds(sid * RPT, RPT)])
        if with_cnt:
            pltpu.sync_copy(cnt_t, cnt_out.at[pl.ds(wid * NP, NP)])

    return k(x, src, dst)


def _combine_body(p_ref, c_ref, x_ref, wl_ref, b_ref, wr_ref, o_ref, *, relu):
    cnt = jnp.maximum(jnp.sum(c_ref[:], axis=0), 1.0)[:, None]
    mean = (p_ref[0] + p_ref[1]) / cnt
    dn = (((1,), (1,)), ((), ()))
    y = lax.dot_general(mean, wl_ref[:], dn,
                        preferred_element_type=jnp.float32)
    y = y + b_ref[:]
    y = y + lax.dot_general(x_ref[:], wr_ref[:], dn,
                            preferred_element_type=jnp.float32)
    o_ref[:] = jnp.maximum(y, 0.0) if relu else y


def _tc_combine(p, c, x, W_l, b_l, W_r, relu):
    R = 1024
    grid = (NP // R,)
    return pl.pallas_call(
        functools.partial(_combine_body, relu=relu),
        grid=grid,
        in_specs=[
            pl.BlockSpec((NC, R, D), lambda i: (0, i, 0)),
            pl.BlockSpec((NW, R), lambda i: (0, i)),
            pl.BlockSpec((R, D), lambda i: (i, 0)),
            pl.BlockSpec((D, D), lambda i: (0, 0)),
            pl.BlockSpec((1, D), lambda i: (0, 0)),
            pl.BlockSpec((D, D), lambda i: (0, 0)),
        ],
        out_specs=pl.BlockSpec((R, D), lambda i: (i, 0)),
        out_shape=jax.ShapeDtypeStruct((NP, D), jnp.float32),
    )(p, c.reshape(NW, NP), x, W_l, b_l.reshape(1, D), W_r)


def kernel(x, edge_index, W1_l, b1_l, W1_r, W2_l, b2_l, W2_r):
    src = edge_index[0]
    dst = edge_index[1]
    x_p = jnp.pad(x, ((0, NP - N), (0, 0)))
    agg1, cnt = _sc_agg(x, src, dst, with_cnt=True)
    h = _tc_combine(agg1, cnt, x_p, W1_l, b1_l, W1_r, relu=True)
    (agg2,) = _sc_agg(h, src, dst, with_cnt=False)
    out = _tc_combine(agg2, cnt, h, W2_l, b2_l, W2_r, relu=False)
    return out[:N]


# speedup vs baseline: 2.7538x; 1.0670x over previous
import functools

import jax
import jax.numpy as jnp
from jax import lax
from jax.experimental import pallas as pl
from jax.experimental.pallas import tpu as pltpu
from jax.experimental.pallas import tpu_sc as plsc

N = 10000
E = 320000
D = 128

NC = 2
NS = 16
NW = NC * NS
EPW = E // NW
C = 80
NCHUNK = EPW // C
NP = 10240
RPT = NP // NS


def _sc_agg(x, src, dst, with_cnt):
    mesh = plsc.VectorSubcoreMesh(core_axis_name="c", subcore_axis_name="s")

    out_type = [jax.ShapeDtypeStruct((NC, NP, D), jnp.float32)]
    scratch = [
        pltpu.VMEM((C,), jnp.int32),
        pltpu.VMEM((C,), jnp.int32),
        pltpu.VMEM((C,), jnp.int32),
        pltpu.VMEM((C,), jnp.int32),
        pltpu.VMEM((C, D), jnp.float32),
        pltpu.VMEM((C, D), jnp.float32),
        pltpu.VMEM_SHARED((NP, D), jnp.float32),
        pltpu.SemaphoreType.DMA,
        pltpu.SemaphoreType.DMA,
        pltpu.SemaphoreType.DMA,
        pltpu.SemaphoreType.DMA,
        pltpu.SemaphoreType.DMA,
        pltpu.SemaphoreType.DMA,
    ]
    if with_cnt:
        out_type.append(jax.ShapeDtypeStruct((NW * NP,), jnp.float32))
        scratch.append(pltpu.VMEM((NP,), jnp.float32))

    @functools.partial(
        pl.kernel, mesh=mesh, out_type=out_type, scratch_types=scratch,
        compiler_params=pltpu.CompilerParams(needs_layout_passes=False))
    def k(x_hbm, src_hbm, dst_hbm, *refs):
        if with_cnt:
            (agg_out, cnt_out, src_a, dst_a, src_b, dst_b, rows_a, rows_b,
             acc, gsem_a, gsem_b, ssem_a, ssem_b, isem_a, isem_b,
             cnt_t) = refs
        else:
            (agg_out, src_a, dst_a, src_b, dst_b, rows_a, rows_b,
             acc, gsem_a, gsem_b, ssem_a, ssem_b, isem_a, isem_b) = refs
        cid = lax.axis_index("c")
        sid = lax.axis_index("s")
        wid = sid * NC + cid

        def zb(i, carry):
            for j in range(D // 16):
                rows_a[i, pl.ds(j * 16, 16)] = jnp.zeros((16,), jnp.float32)
            return carry
        lax.fori_loop(0, C, zb, 0)
        for t in range(RPT // C):
            pltpu.sync_copy(rows_a, acc.at[pl.ds(sid * RPT + t * C, C)])
        if with_cnt:
            def zc(i, carry):
                cnt_t[pl.ds(i * 16, 16)] = jnp.zeros((16,), jnp.float32)
                return carry
            lax.fori_loop(0, NP // 16, zc, 0)
        plsc.subcore_barrier()

        base0 = wid * EPW

        def stage(i, src_c, dst_c, sem):
            pltpu.async_copy(src_hbm.at[pl.ds(base0 + i * C, C)], src_c, sem)
            pltpu.async_copy(dst_hbm.at[pl.ds(base0 + i * C, C)], dst_c, sem)

        def iwait(i, src_c, dst_c, sem):
            pltpu.make_async_copy(
                src_hbm.at[pl.ds(base0 + i * C, C)], src_c, sem).wait()
            pltpu.make_async_copy(
                dst_hbm.at[pl.ds(base0 + i * C, C)], dst_c, sem).wait()

        def gather(buf, src_c, sem):
            pltpu.async_copy(x_hbm.at[src_c], buf, sem)

        def gwait(buf, src_c, sem):
            pltpu.make_async_copy(x_hbm.at[src_c], buf, sem).wait()

        def scat(buf, dst_c, sem):
            pltpu.async_copy(buf, acc.at[dst_c], sem, add=True)

        def swait(buf, dst_c, sem):
            pltpu.make_async_copy(buf, acc.at[dst_c], sem).wait()

        def count(dst_c):
            if with_cnt:
                ones16 = jnp.ones((16,), jnp.float32)
                for k2 in range(C // 16):
                    idx = dst_c[pl.ds(k2 * 16, 16)]
                    plsc.addupdate_scatter(cnt_t, [idx], ones16)

        stage(0, src_a, dst_a, isem_a)
        iwait(0, src_a, dst_a, isem_a)
        gather(rows_a, src_a, gsem_a)
        stage(1, src_b, dst_b, isem_b)

        def body(j, carry):
            i0 = 2 * j
            iwait(i0 + 1, src_b, dst_b, isem_b)
            gather(rows_b, src_b, gsem_b)
            gwait(rows_a, src_a, gsem_a)
            scat(rows_a, dst_a, ssem_a)
            count(dst_a)
            swait(rows_a, dst_a, ssem_a)
            stage(i0 + 2, src_a, dst_a, isem_a)
            gwait(rows_b, src_b, gsem_b)
            scat(rows_b, dst_b, ssem_b)
            count(dst_b)
            iwait(i0 + 2, src_a, dst_a, isem_a)
            gather(rows_a, src_a, gsem_a)
            swait(rows_b, dst_b, ssem_b)
            @pl.when(j + 1 < NCHUNK // 2)
            def _():
                stage(i0 + 3, src_b, dst_b, isem_b)
            return carry
        lax.fori_loop(0, NCHUNK // 2, body, 0)
        gwait(rows_a, src_a, gsem_a)
        pltpu.sync_copy(rows_a, acc.at[dst_a], add=True)
        count(dst_a)
        plsc.subcore_barrier()

        pltpu.sync_copy(acc.at[pl.ds(sid * RPT, RPT)],
                        agg_out.at[cid, pl.ds(sid * RPT, RPT)])
        if with_cnt:
            pltpu.sync_copy(cnt_t, cnt_out.at[pl.ds(wid * NP, NP)])

    return k(x, src, dst)


def _combine_body(p_ref, c_ref, x_ref, wl_ref, b_ref, wr_ref, o_ref, *, relu):
    cnt = jnp.maximum(jnp.sum(c_ref[:], axis=0), 1.0)[:, None]
    mean = (p_ref[0] + p_ref[1]) / cnt
    dn = (((1,), (1,)), ((), ()))
    y = lax.dot_general(mean, wl_ref[:], dn,
                        preferred_element_type=jnp.float32)
    y = y + b_ref[:]
    y = y + lax.dot_general(x_ref[:], wr_ref[:], dn,
                            preferred_element_type=jnp.float32)
    o_ref[:] = jnp.maximum(y, 0.0) if relu else y


def _tc_combine(p, c, x, W_l, b_l, W_r, relu):
    R = 1024
    grid = (NP // R,)
    return pl.pallas_call(
        functools.partial(_combine_body, relu=relu),
        grid=grid,
        in_specs=[
            pl.BlockSpec((NC, R, D), lambda i: (0, i, 0)),
            pl.BlockSpec((NW, R), lambda i: (0, i)),
            pl.BlockSpec((R, D), lambda i: (i, 0)),
            pl.BlockSpec((D, D), lambda i: (0, 0)),
            pl.BlockSpec((1, D), lambda i: (0, 0)),
            pl.BlockSpec((D, D), lambda i: (0, 0)),
        ],
        out_specs=pl.BlockSpec((R, D), lambda i: (i, 0)),
        out_shape=jax.ShapeDtypeStruct((NP, D), jnp.float32),
    )(p, c.reshape(NW, NP), x, W_l, b_l.reshape(1, D), W_r)


def kernel(x, edge_index, W1_l, b1_l, W1_r, W2_l, b2_l, W2_r):
    src = edge_index[0]
    dst = edge_index[1]
    x_p = jnp.pad(x, ((0, NP - N), (0, 0)))
    agg1, cnt = _sc_agg(x, src, dst, with_cnt=True)
    h = _tc_combine(agg1, cnt, x_p, W1_l, b1_l, W1_r, relu=True)
    (agg2,) = _sc_agg(h, src, dst, with_cnt=False)
    out = _tc_combine(agg2, cnt, h, W2_l, b2_l, W2_r, relu=False)
    return out[:N]
